# baseline (speedup 1.0000x reference)
import functools

import jax
import jax.numpy as jnp
from jax import lax
from jax.experimental import pallas as pl
from jax.experimental.pallas import tpu as pltpu
from jax.experimental.pallas import tpu_sc as plsc

N_CODES = 8192
E_DIM = 256
BETA = 0.25
N_TOKENS = 8192
TILE_T = 128
SPATIAL = 1024


def _vq_tc_body(z_ref, emb_ref, enc_ref, idx_ref, loss_ref, perp_ref,
                counts_ref, dsum_ref, b_ref, iotaf_ref):
    step = pl.program_id(0)
    nsteps = pl.num_programs(0)

    emb = emb_ref[...]

    @pl.when(step == 0)
    def _init():
        counts_ref[...] = jnp.zeros_like(counts_ref)
        dsum_ref[...] = jnp.zeros_like(dsum_ref)
        ones_row = jnp.ones((1, E_DIM), jnp.float32)
        b_ref[...] = lax.dot_general(
            ones_row, emb * emb, (((1,), (1,)), ((), ())),
            preferred_element_type=jnp.float32)
        iotaf_ref[...] = lax.broadcasted_iota(
            jnp.int32, (1, N_CODES), 1).astype(jnp.float32)

    z_t = jnp.transpose(z_ref[0], (1, 0))
    zsq = z_t * z_t
    ones_col = jnp.ones((1, E_DIM), jnp.float32)
    a = lax.dot_general(zsq, ones_col, (((1,), (1,)), ((), ())),
                        preferred_element_type=jnp.float32)
    z2 = z_t + z_t
    s2 = lax.dot_general(z2, emb, (((1,), (1,)), ((), ())),
                         preferred_element_type=jnp.float32)
    d = (a + b_ref[...]) - s2
    dmin2 = jnp.min(d, axis=1, keepdims=True)
    iota = iotaf_ref[...]
    cand = jnp.where(d == dmin2, iota, jnp.float32(N_CODES))
    idxf = jnp.min(cand, axis=1, keepdims=True)
    onehot = (iota == idxf).astype(jnp.float32)
    enc_ref[...] = onehot
    idx_ref[...] = idxf.astype(jnp.int32)
    ones_t = jnp.ones((1, TILE_T), jnp.float32)
    counts_ref[...] += lax.dot_general(
        ones_t, onehot, (((1,), (0,)), ((), ())),
        preferred_element_type=jnp.float32)
    dsum_ref[...] += jnp.sum(dmin2).reshape(1, 1)

    @pl.when(step == nsteps - 1)
    def _finalize():
        total = jnp.float32(N_TOKENS * E_DIM)
        loss_ref[...] = (1.0 + BETA) * (dsum_ref[...] / total)
        p = counts_ref[...] * (1.0 / N_TOKENS)
        perp_ref[...] = jnp.exp(-jnp.sum(p * jnp.log(p + 1e-10))).reshape(1, 1)


def _vq_distances_argmin(z3, emb_w):
    tiles_per_b = SPATIAL // TILE_T
    grid = (N_TOKENS // TILE_T,)
    return pl.pallas_call(
        _vq_tc_body,
        grid=grid,
        in_specs=[
            pl.BlockSpec((1, E_DIM, TILE_T),
                         lambda i: (i // tiles_per_b, 0, i % tiles_per_b)),
            pl.BlockSpec((N_CODES, E_DIM), lambda i: (0, 0)),
        ],
        out_specs=[
            pl.BlockSpec((TILE_T, N_CODES), lambda i: (i, 0)),
            pl.BlockSpec((TILE_T, 1), lambda i: (i, 0)),
            pl.BlockSpec((1, 1), lambda i: (0, 0)),
            pl.BlockSpec((1, 1), lambda i: (0, 0)),
        ],
        out_shape=[
            jax.ShapeDtypeStruct((N_TOKENS, N_CODES), jnp.float32),
            jax.ShapeDtypeStruct((N_TOKENS, 1), jnp.int32),
            jax.ShapeDtypeStruct((1, 1), jnp.float32),
            jax.ShapeDtypeStruct((1, 1), jnp.float32),
        ],
        scratch_shapes=[
            pltpu.VMEM((1, N_CODES), jnp.float32),
            pltpu.VMEM((1, 1), jnp.float32),
            pltpu.VMEM((1, N_CODES), jnp.float32),
            pltpu.VMEM((1, N_CODES), jnp.float32),
        ],
        compiler_params=pltpu.CompilerParams(
            dimension_semantics=("arbitrary",),
        ),
    )(z3, emb_w)


@functools.lru_cache(maxsize=1)
def _make_sc_gather():
    info = plsc.get_sparse_core_info()
    nc, ns = info.num_cores, info.num_subcores
    nw = nc * ns
    b_per_w = N_TOKENS // nw
    mesh = plsc.VectorSubcoreMesh(core_axis_name="c", subcore_axis_name="s")

    @functools.partial(
        pl.kernel, mesh=mesh,
        out_type=jax.ShapeDtypeStruct((N_TOKENS, E_DIM), jnp.float32),
        scratch_types=[
            pltpu.VMEM((b_per_w,), jnp.int32),
            pltpu.VMEM((b_per_w, E_DIM), jnp.float32),
            pltpu.SemaphoreType.DMA,
        ],
    )
    def gather(table_hbm, idx_hbm, out_hbm, idx_v, rows_v, sem):
        wid = lax.axis_index("s") * nc + lax.axis_index("c")
        base = wid * b_per_w
        pltpu.sync_copy(idx_hbm.at[pl.ds(base, b_per_w)], idx_v)
        pltpu.async_copy(table_hbm.at[idx_v], rows_v, sem).wait()
        pltpu.sync_copy(rows_v, out_hbm.at[pl.ds(base, b_per_w)])

    return gather


def kernel(z, emb_w):
    z3 = z.reshape(z.shape[0], E_DIM, SPATIAL)
    enc, idx2, loss, perp = _vq_distances_argmin(z3, emb_w)
    z_q = _make_sc_gather()(emb_w, idx2.reshape(-1))
    z_q_out = jnp.transpose(z_q.reshape(z.shape[0], 32, 32, E_DIM),
                            (0, 3, 1, 2))
    return (z_q_out, loss[0, 0], perp[0, 0], enc, idx2)


# speedup vs baseline: 1.5142x; 1.5142x over previous
import functools

import jax
import jax.numpy as jnp
from jax import lax
from jax.experimental import pallas as pl
from jax.experimental.pallas import tpu as pltpu
from jax.experimental.pallas import tpu_sc as plsc

N_CODES = 8192
E_DIM = 256
BETA = 0.25
N_TOKENS = 8192
TILE_T = 512
SPATIAL = 1024


def _vq_tc_body(z_ref, emb_ref, enc_ref, idx_ref, loss_ref, perp_ref,
                counts_ref, dsum_ref, b_ref, iotaf_ref):
    step = pl.program_id(0)
    nsteps = pl.num_programs(0)

    emb = emb_ref[...]

    @pl.when(step == 0)
    def _init():
        counts_ref[...] = jnp.zeros_like(counts_ref)
        dsum_ref[...] = jnp.zeros_like(dsum_ref)
        ones_row = jnp.ones((1, E_DIM), jnp.float32)
        b_ref[...] = lax.dot_general(
            ones_row, emb * emb, (((1,), (1,)), ((), ())),
            preferred_element_type=jnp.float32)
        iotaf_ref[...] = lax.broadcasted_iota(
            jnp.int32, (1, N_CODES), 1).astype(jnp.float32)

    z_t = jnp.transpose(z_ref[0], (1, 0))
    zsq = z_t * z_t
    ones_col = jnp.ones((1, E_DIM), jnp.float32)
    a = lax.dot_general(zsq, ones_col, (((1,), (1,)), ((), ())),
                        preferred_element_type=jnp.float32)
    z2 = z_t + z_t
    s2 = lax.dot_general(z2, emb, (((1,), (1,)), ((), ())),
                         preferred_element_type=jnp.float32)
    d = (a + b_ref[...]) - s2
    dmin2 = jnp.min(d, axis=1, keepdims=True)
    iota = iotaf_ref[...]
    cand = jnp.where(d == dmin2, iota, jnp.float32(N_CODES))
    idxf = jnp.min(cand, axis=1, keepdims=True)
    onehot = (iota == idxf).astype(jnp.float32)
    enc_ref[...] = onehot
    idx_ref[...] = idxf.astype(jnp.int32)
    ones_t = jnp.ones((1, TILE_T), jnp.float32)
    counts_ref[...] += lax.dot_general(
        ones_t, onehot, (((1,), (0,)), ((), ())),
        preferred_element_type=jnp.float32)
    dsum_ref[...] += jnp.sum(dmin2).reshape(1, 1)

    @pl.when(step == nsteps - 1)
    def _finalize():
        total = jnp.float32(N_TOKENS * E_DIM)
        loss_ref[...] = (1.0 + BETA) * (dsum_ref[...] / total)
        p = counts_ref[...] * (1.0 / N_TOKENS)
        perp_ref[...] = jnp.exp(-jnp.sum(p * jnp.log(p + 1e-10))).reshape(1, 1)


def _vq_distances_argmin(z3, emb_w):
    tiles_per_b = SPATIAL // TILE_T
    grid = (N_TOKENS // TILE_T,)
    return pl.pallas_call(
        _vq_tc_body,
        grid=grid,
        in_specs=[
            pl.BlockSpec((1, E_DIM, TILE_T),
                         lambda i: (i // tiles_per_b, 0, i % tiles_per_b)),
            pl.BlockSpec((N_CODES, E_DIM), lambda i: (0, 0)),
        ],
        out_specs=[
            pl.BlockSpec((TILE_T, N_CODES), lambda i: (i, 0)),
            pl.BlockSpec((TILE_T, 1), lambda i: (i, 0)),
            pl.BlockSpec((1, 1), lambda i: (0, 0)),
            pl.BlockSpec((1, 1), lambda i: (0, 0)),
        ],
        out_shape=[
            jax.ShapeDtypeStruct((N_TOKENS, N_CODES), jnp.float32),
            jax.ShapeDtypeStruct((N_TOKENS, 1), jnp.int32),
            jax.ShapeDtypeStruct((1, 1), jnp.float32),
            jax.ShapeDtypeStruct((1, 1), jnp.float32),
        ],
        scratch_shapes=[
            pltpu.VMEM((1, N_CODES), jnp.float32),
            pltpu.VMEM((1, 1), jnp.float32),
            pltpu.VMEM((1, N_CODES), jnp.float32),
            pltpu.VMEM((1, N_CODES), jnp.float32),
        ],
        compiler_params=pltpu.CompilerParams(
            dimension_semantics=("arbitrary",),
            vmem_limit_bytes=100 * 1024 * 1024,
        ),
    )(z3, emb_w)


@functools.lru_cache(maxsize=1)
def _make_sc_gather():
    info = plsc.get_sparse_core_info()
    nc, ns = info.num_cores, info.num_subcores
    nw = nc * ns
    b_per_w = N_TOKENS // nw
    mesh = plsc.VectorSubcoreMesh(core_axis_name="c", subcore_axis_name="s")

    @functools.partial(
        pl.kernel, mesh=mesh,
        out_type=jax.ShapeDtypeStruct((N_TOKENS, E_DIM), jnp.float32),
        scratch_types=[
            pltpu.VMEM((b_per_w,), jnp.int32),
            pltpu.VMEM((b_per_w, E_DIM), jnp.float32),
            pltpu.SemaphoreType.DMA,
        ],
    )
    def gather(table_hbm, idx_hbm, out_hbm, idx_v, rows_v, sem):
        wid = lax.axis_index("s") * nc + lax.axis_index("c")
        base = wid * b_per_w
        pltpu.sync_copy(idx_hbm.at[pl.ds(base, b_per_w)], idx_v)
        pltpu.async_copy(table_hbm.at[idx_v], rows_v, sem).wait()
        pltpu.sync_copy(rows_v, out_hbm.at[pl.ds(base, b_per_w)])

    return gather


def kernel(z, emb_w):
    z3 = z.reshape(z.shape[0], E_DIM, SPATIAL)
    enc, idx2, loss, perp = _vq_distances_argmin(z3, emb_w)
    z_q = _make_sc_gather()(emb_w, idx2.reshape(-1))
    z_q_out = jnp.transpose(z_q.reshape(z.shape[0], 32, 32, E_DIM),
                            (0, 3, 1, 2))
    return (z_q_out, loss[0, 0], perp[0, 0], enc, idx2)


# speedup vs baseline: 1.5802x; 1.0436x over previous
import functools

import jax
import jax.numpy as jnp
from jax import lax
from jax.experimental import pallas as pl
from jax.experimental.pallas import tpu as pltpu
from jax.experimental.pallas import tpu_sc as plsc

N_CODES = 8192
E_DIM = 256
BETA = 0.25
N_TOKENS = 8192
TILE_T = 512
SPATIAL = 1024


def _vq_tc_body(z_ref, emb_ref, enc_ref, idx_ref, loss_ref, perp_ref,
                counts_ref, dsum_ref, b_ref, iotaf_ref):
    step = pl.program_id(0)
    nsteps = pl.num_programs(0)

    @pl.when(step == 0)
    def _init():
        counts_ref[...] = jnp.zeros_like(counts_ref)
        dsum_ref[...] = jnp.zeros_like(dsum_ref)
        emb = emb_ref[...]
        ones_row = jnp.ones((1, E_DIM), jnp.float32)
        b_ref[...] = lax.dot_general(
            ones_row, emb * emb, (((1,), (1,)), ((), ())),
            preferred_element_type=jnp.float32)
        iotaf_ref[...] = lax.broadcasted_iota(
            jnp.int32, (1, N_CODES), 1).astype(jnp.float32)

    z_t = jnp.transpose(z_ref[0], (1, 0))
    zsq = z_t * z_t
    ones_col = jnp.ones((1, E_DIM), jnp.float32)
    a = lax.dot_general(zsq, ones_col, (((1,), (1,)), ((), ())),
                        preferred_element_type=jnp.float32)
    z2 = z_t + z_t
    s2 = lax.dot_general(z2, emb_ref[...], (((1,), (1,)), ((), ())),
                         preferred_element_type=jnp.float32)
    b = b_ref[...]
    dmin2 = jnp.min((a + b) - s2, axis=1, keepdims=True)
    iota = iotaf_ref[...]
    cand = jnp.where(((a + b) - s2) == dmin2, iota, jnp.float32(N_CODES))
    idxf = jnp.min(cand, axis=1, keepdims=True)
    onehot = (iota == idxf).astype(jnp.float32)
    enc_ref[...] = onehot
    idx_ref[...] = idxf.astype(jnp.int32)
    ones_t = jnp.ones((1, TILE_T), jnp.float32)
    counts_ref[...] += lax.dot_general(
        ones_t, onehot, (((1,), (0,)), ((), ())),
        preferred_element_type=jnp.float32)
    dsum_ref[...] += jnp.sum(dmin2).reshape(1, 1)

    @pl.when(step == nsteps - 1)
    def _finalize():
        total = jnp.float32(N_TOKENS * E_DIM)
        loss_ref[...] = (1.0 + BETA) * (dsum_ref[...] / total)
        p = counts_ref[...] * (1.0 / N_TOKENS)
        perp_ref[...] = jnp.exp(-jnp.sum(p * jnp.log(p + 1e-10))).reshape(1, 1)


def _vq_distances_argmin(z3, emb_w):
    tiles_per_b = SPATIAL // TILE_T
    grid = (N_TOKENS // TILE_T,)
    return pl.pallas_call(
        _vq_tc_body,
        grid=grid,
        in_specs=[
            pl.BlockSpec((1, E_DIM, TILE_T),
                         lambda i: (i // tiles_per_b, 0, i % tiles_per_b)),
            pl.BlockSpec((N_CODES, E_DIM), lambda i: (0, 0)),
        ],
        out_specs=[
            pl.BlockSpec((TILE_T, N_CODES), lambda i: (i, 0)),
            pl.BlockSpec((TILE_T, 1), lambda i: (i, 0)),
            pl.BlockSpec((1, 1), lambda i: (0, 0)),
            pl.BlockSpec((1, 1), lambda i: (0, 0)),
        ],
        out_shape=[
            jax.ShapeDtypeStruct((N_TOKENS, N_CODES), jnp.float32),
            jax.ShapeDtypeStruct((N_TOKENS, 1), jnp.int32),
            jax.ShapeDtypeStruct((1, 1), jnp.float32),
            jax.ShapeDtypeStruct((1, 1), jnp.float32),
        ],
        scratch_shapes=[
            pltpu.VMEM((1, N_CODES), jnp.float32),
            pltpu.VMEM((1, 1), jnp.float32),
            pltpu.VMEM((1, N_CODES), jnp.float32),
            pltpu.VMEM((1, N_CODES), jnp.float32),
        ],
        compiler_params=pltpu.CompilerParams(
            dimension_semantics=("arbitrary",),
            vmem_limit_bytes=100 * 1024 * 1024,
        ),
    )(z3, emb_w)


@functools.lru_cache(maxsize=1)
def _make_sc_gather():
    info = plsc.get_sparse_core_info()
    nc, ns = info.num_cores, info.num_subcores
    nw = nc * ns
    b_per_w = N_TOKENS // nw
    mesh = plsc.VectorSubcoreMesh(core_axis_name="c", subcore_axis_name="s")

    @functools.partial(
        pl.kernel, mesh=mesh,
        out_type=jax.ShapeDtypeStruct((N_TOKENS, E_DIM), jnp.float32),
        scratch_types=[
            pltpu.VMEM((b_per_w,), jnp.int32),
            pltpu.VMEM((b_per_w, E_DIM), jnp.float32),
            pltpu.SemaphoreType.DMA,
        ],
    )
    def gather(table_hbm, idx_hbm, out_hbm, idx_v, rows_v, sem):
        wid = lax.axis_index("s") * nc + lax.axis_index("c")
        base = wid * b_per_w
        pltpu.sync_copy(idx_hbm.at[pl.ds(base, b_per_w)], idx_v)
        pltpu.async_copy(table_hbm.at[idx_v], rows_v, sem).wait()
        pltpu.sync_copy(rows_v, out_hbm.at[pl.ds(base, b_per_w)])

    return gather


def kernel(z, emb_w):
    z3 = z.reshape(z.shape[0], E_DIM, SPATIAL)
    enc, idx2, loss, perp = _vq_distances_argmin(z3, emb_w)
    z_q = _make_sc_gather()(emb_w, idx2.reshape(-1))
    z_q_out = jnp.transpose(z_q.reshape(z.shape[0], 32, 32, E_DIM),
                            (0, 3, 1, 2))
    return (z_q_out, loss[0, 0], perp[0, 0], enc, idx2)
